# baseline (speedup 1.0000x reference)
import jax
import jax.numpy as jnp
from jax import lax
from jax.experimental import pallas as pl
from jax.experimental.pallas import tpu as pltpu

K = 15
C = 10
N = 20000
B = 64
G = 160
NG = N // G
THRESHOLD = 0.7
NEG = -3.0e38


def _boxmax_body(x_ref, out_ref):
    acc = x_ref[0]
    for c in range(1, C):
        acc = jnp.maximum(acc, x_ref[c])
    out_ref[...] = acc


def _select_boxes_body(bm_ref, selbox_ref):
    bm = bm_ref[...]
    gmax = jnp.max(bm, axis=2)
    gid_iota = lax.broadcasted_iota(jnp.int32, (B, NG), 1)

    gids = []
    gm = gmax
    m = None
    for _ in range(K):
        m = jnp.max(gm, axis=1, keepdims=True)
        gi = jnp.min(jnp.where(gm == m, gid_iota, NG), axis=1,
                     keepdims=True)
        gids.append(gi)
        gm = jnp.where(gid_iota == gi, NEG, gm)
    t15 = m

    gsel = jnp.concatenate(gids, axis=1)
    g_iota = lax.broadcasted_iota(jnp.int32, (B, K, NG), 2)
    onehot = (g_iota == gsel[:, :, None]).astype(jnp.float32)
    cand = lax.dot_general(onehot, bm,
                           (((2,), (1,)), ((0,), (0,))),
                           precision=lax.Precision.HIGHEST,
                           preferred_element_type=jnp.float32)
    boxid = gsel[:, :, None] * G + lax.broadcasted_iota(
        jnp.int32, (B, K, G), 2)

    cand = jnp.where(cand >= t15[:, :, None], cand, NEG)

    big = jnp.int32(N)
    for k in range(K):
        m1 = jnp.max(cand, axis=1)
        m = jnp.max(m1, axis=1, keepdims=True)
        eq = cand == m[:, :, None]
        bi1 = jnp.min(jnp.where(eq, boxid, big), axis=1)
        bi = jnp.min(bi1, axis=1, keepdims=True)
        selbox_ref[:, pl.ds(k, 1)] = bi
        cand = jnp.where(boxid == bi[:, :, None], NEG, cand)


def _dyn_lane_shift(h, l3, width, step2):
    a = l3 // 16
    s = l3 % 16
    h1 = jnp.zeros_like(h)
    for o in range(8):
        r = h if o == 0 else jnp.roll(h, -16 * o, axis=2)
        h1 = jnp.where(a == o, r, h1)
    h2 = jnp.zeros_like(h)
    for o in range(0, 16, step2):
        r = h1 if o == 0 else jnp.roll(h1, -o, axis=2)
        h2 = jnp.where(s == o, r, h2)
    return h2[:, :, :width]


def _finalize_body(sel_s, selbox_ref, flat_hbm, box_hbm, out_ref,
                   e_ref, b_ref, sem1, sem2):
    def _descr(i):
        b = i // K
        k = i - b * K
        n = sel_s[b, k]
        st = b * (N * C) + n * C
        cp1 = pltpu.make_async_copy(
            flat_hbm.at[pl.ds(st // 1024, 2)], e_ref.at[i], sem1)
        bst = b * (N * 4) + n * 4
        cp2 = pltpu.make_async_copy(
            box_hbm.at[pl.ds(bst // 1024, 1)], b_ref.at[i], sem2)
        return cp1, cp2

    def _fire(i, _):
        cp1, cp2 = _descr(i)
        cp1.start()
        cp2.start()
        return 0

    def _drain(i, _):
        cp1, cp2 = _descr(i)
        cp1.wait()
        cp2.wait()
        return 0

    lax.fori_loop(0, B * K, _fire, 0, unroll=8)
    lax.fori_loop(0, B * K, _drain, 0, unroll=8)

    selbox = selbox_ref[...]
    bcol = lax.broadcasted_iota(jnp.int32, (B, K), 0)
    st = bcol * (N * C) + selbox * C
    q = st % 1024
    u3 = (q // 128)[:, :, None]
    l3 = (q % 128)[:, :, None]

    ew = e_ref[...].reshape(B, K, 16, 128)
    sub_iota = lax.broadcasted_iota(jnp.int32, (B, K, 16, 128), 2)
    u4 = u3[:, :, :, None]
    h0 = jnp.sum(jnp.where(sub_iota == u4, ew, 0.0), axis=2)
    h1 = jnp.sum(jnp.where(sub_iota == u4 + 1, ew, 0.0), axis=2)
    g0 = _dyn_lane_shift(h0, l3, C, 2)
    g1 = _dyn_lane_shift(h1, l3, C, 2)
    j10 = lax.broadcasted_iota(jnp.int32, (B, K, C), 2)
    el3 = jnp.where(j10 < 128 - l3, g0, g1)

    bst = bcol * (N * 4) + selbox * 4
    qb = bst % 1024
    ub4 = (qb // 128)[:, :, None, None]
    lb3 = (qb % 128)[:, :, None]
    bw = b_ref[...].reshape(B, K, 8, 128)
    sub8 = lax.broadcasted_iota(jnp.int32, (B, K, 8, 128), 2)
    hb = jnp.sum(jnp.where(sub8 == ub4, bw, 0.0), axis=2)
    b43 = _dyn_lane_shift(hb, lb3, 4, 4)
    fi = selbox[:, :, None] * C + lax.broadcasted_iota(
        jnp.int32, (B, K, C), 2)

    vals = el3
    big = jnp.int32(N * C)
    out_rows = []
    keys = []
    for r in range(K):
        m = jnp.max(jnp.max(vals, axis=1), axis=1, keepdims=True)
        eq = vals == m[:, :, None]
        fr = jnp.min(jnp.min(jnp.where(eq, fi, big), axis=1),
                     axis=1, keepdims=True)
        vals = jnp.where(fi == fr[:, :, None], NEG, vals)
        score = 1.0 / (1.0 + jnp.exp(-m))
        use = jnp.where(score >= THRESHOLD, 1.0, 0.0)
        bi = fr // C
        pmask = (selbox == bi).astype(jnp.float32)[:, :, None]
        lrow = jnp.sum(pmask * el3, axis=1)
        brow = jnp.sum(pmask * b43, axis=1)
        prow = 1.0 / (1.0 + jnp.exp(-lrow))
        row = jnp.concatenate([brow, prow], axis=1) * use
        out_rows.append(row[:, None, :])
        keys.append((row[:, 0:1]))
    player = jnp.concatenate(out_rows, axis=1)
    kmat = jnp.concatenate(keys, axis=1)

    slot_iota = lax.broadcasted_iota(jnp.int32, (B, K), 1)
    for i in range(K):
        m = jnp.max(kmat, axis=1, keepdims=True)
        p = jnp.min(jnp.where(kmat == m, slot_iota, K), axis=1,
                    keepdims=True)
        sel = (slot_iota == p).astype(jnp.float32)[:, :, None]
        out_ref[:, pl.ds(i, 1), :] = jnp.sum(sel * player, axis=1,
                                             keepdims=True)
        kmat = jnp.where(slot_iota == p, NEG, kmat)


@jax.jit
def kernel(logits, boxes):
    x_t = logits.transpose(2, 0, 1)
    flat3d = logits.reshape(12500, 8, 128)
    box3d = boxes.reshape(5000, 8, 128)

    boxmax = pl.pallas_call(
        _boxmax_body,
        grid=(B // 8,),
        in_specs=[pl.BlockSpec((C, 8, N), lambda i: (0, i, 0))],
        out_specs=pl.BlockSpec((8, N), lambda i: (i, 0)),
        out_shape=jax.ShapeDtypeStruct((B, N), jnp.float32),
    )(x_t)

    bm3 = boxmax.reshape(B, NG, G)

    selbox = pl.pallas_call(
        _select_boxes_body,
        grid=(1,),
        in_specs=[pl.BlockSpec((B, NG, G), lambda i: (0, 0, 0))],
        out_specs=pl.BlockSpec((B, K), lambda i: (0, 0)),
        out_shape=jax.ShapeDtypeStruct((B, K), jnp.int32),
    )(bm3)

    out = pl.pallas_call(
        _finalize_body,
        grid_spec=pltpu.PrefetchScalarGridSpec(
            num_scalar_prefetch=1,
            grid=(1,),
            in_specs=[
                pl.BlockSpec((B, K), lambda i, s: (0, 0)),
                pl.BlockSpec(memory_space=pl.ANY),
                pl.BlockSpec(memory_space=pl.ANY),
            ],
            out_specs=pl.BlockSpec((B, K, 14), lambda i, s: (0, 0, 0)),
            scratch_shapes=[
                pltpu.VMEM((B * K, 2, 8, 128), jnp.float32),
                pltpu.VMEM((B * K, 1, 8, 128), jnp.float32),
                pltpu.SemaphoreType.DMA,
                pltpu.SemaphoreType.DMA,
            ],
        ),
        out_shape=jax.ShapeDtypeStruct((B, K, 14), jnp.float32),
    )(selbox, selbox, flat3d, box3d)
    return out


# speedup vs baseline: 3.9529x; 3.9529x over previous
import jax
import jax.numpy as jnp
from jax import lax
from jax.experimental import pallas as pl
from jax.experimental.pallas import tpu as pltpu

K = 15
C = 10
N = 20000
B = 64
G = 160
NG = N // G
THRESHOLD = 0.7
NEG = -3.0e38


def _boxmax_body(x_ref, out_ref):
    acc = x_ref[0]
    for c in range(1, C):
        acc = jnp.maximum(acc, x_ref[c])
    out_ref[...] = acc


def _select_boxes_body(bm_ref, selbox_ref):
    bm = bm_ref[...]
    gmax = jnp.max(bm, axis=2)
    gid_iota = lax.broadcasted_iota(jnp.int32, (B, NG), 1)

    gids = []
    gm = gmax
    m = None
    for _ in range(K):
        m = jnp.max(gm, axis=1, keepdims=True)
        gi = jnp.min(jnp.where(gm == m, gid_iota, NG), axis=1,
                     keepdims=True)
        gids.append(gi)
        gm = jnp.where(gid_iota == gi, NEG, gm)
    t15 = m

    gsel = jnp.concatenate(gids, axis=1)
    g_iota = lax.broadcasted_iota(jnp.int32, (B, K, NG), 2)
    onehot = (g_iota == gsel[:, :, None]).astype(jnp.float32)
    cand = lax.dot_general(onehot, bm,
                           (((2,), (1,)), ((0,), (0,))),
                           precision=lax.Precision.HIGHEST,
                           preferred_element_type=jnp.float32)
    boxid = gsel[:, :, None] * G + lax.broadcasted_iota(
        jnp.int32, (B, K, G), 2)

    cand = jnp.where(cand >= t15[:, :, None], cand, NEG)

    big = jnp.int32(N)
    for k in range(K):
        m1 = jnp.max(cand, axis=1)
        m = jnp.max(m1, axis=1, keepdims=True)
        eq = cand == m[:, :, None]
        bi1 = jnp.min(jnp.where(eq, boxid, big), axis=1)
        bi = jnp.min(bi1, axis=1, keepdims=True)
        selbox_ref[:, pl.ds(k, 1)] = bi
        cand = jnp.where(boxid == bi[:, :, None], NEG, cand)


def _gather_body(sel_s, x_ref, bx_ref, eo_ref, bo_ref):
    i = pl.program_id(0)
    u = sel_s[i, 2]
    sub_c = lax.broadcasted_iota(jnp.int32, (C, 8, 128), 1)
    eo_ref[...] = jnp.sum(jnp.where(sub_c == u, x_ref[...], 0.0),
                          axis=1)[None]
    sub_b = lax.broadcasted_iota(jnp.int32, (4, 8, 128), 1)
    bo_ref[...] = jnp.sum(jnp.where(sub_b == u, bx_ref[...], 0.0),
                          axis=1)[None]


def _finalize_body(selbox_ref, eslab_ref, bslab_ref, out_ref):
    selbox = selbox_ref[...]
    bcol = lax.broadcasted_iota(jnp.int32, (B, K), 0)
    l4 = (selbox % 128)[:, :, None, None]

    ew = eslab_ref[...].reshape(B, K, C, 128)
    lane4 = lax.broadcasted_iota(jnp.int32, (B, K, C, 128), 3)
    el3 = jnp.sum(jnp.where(lane4 == l4, ew, 0.0), axis=3)

    bw = bslab_ref[...].reshape(B, K, 4, 128)
    lb4 = lax.broadcasted_iota(jnp.int32, (B, K, 4, 128), 3)
    b43 = jnp.sum(jnp.where(lb4 == l4, bw, 0.0), axis=3)

    fi = selbox[:, :, None] * C + lax.broadcasted_iota(
        jnp.int32, (B, K, C), 2)

    vals = el3
    big = jnp.int32(N * C)
    out_rows = []
    keys = []
    for r in range(K):
        m = jnp.max(jnp.max(vals, axis=1), axis=1, keepdims=True)
        eq = vals == m[:, :, None]
        fr = jnp.min(jnp.min(jnp.where(eq, fi, big), axis=1),
                     axis=1, keepdims=True)
        vals = jnp.where(fi == fr[:, :, None], NEG, vals)
        score = 1.0 / (1.0 + jnp.exp(-m))
        use = jnp.where(score >= THRESHOLD, 1.0, 0.0)
        bi = fr // C
        pmask = (selbox == bi).astype(jnp.float32)[:, :, None]
        lrow = jnp.sum(pmask * el3, axis=1)
        brow = jnp.sum(pmask * b43, axis=1)
        prow = 1.0 / (1.0 + jnp.exp(-lrow))
        row = jnp.concatenate([brow, prow], axis=1) * use
        out_rows.append(row[:, None, :])
        keys.append(row[:, 0:1])
    player = jnp.concatenate(out_rows, axis=1)
    kmat = jnp.concatenate(keys, axis=1)

    slot_iota = lax.broadcasted_iota(jnp.int32, (B, K), 1)
    for i in range(K):
        m = jnp.max(kmat, axis=1, keepdims=True)
        p = jnp.min(jnp.where(kmat == m, slot_iota, K), axis=1,
                    keepdims=True)
        sel = (slot_iota == p).astype(jnp.float32)[:, :, None]
        out_ref[:, pl.ds(i, 1), :] = jnp.sum(sel * player, axis=1,
                                             keepdims=True)
        kmat = jnp.where(slot_iota == p, NEG, kmat)


@jax.jit
def kernel(logits, boxes):
    x_t = logits.transpose(2, 0, 1)
    bx_t = boxes.transpose(2, 0, 1)

    boxmax = pl.pallas_call(
        _boxmax_body,
        grid=(B // 8,),
        in_specs=[pl.BlockSpec((C, 8, N), lambda i: (0, i, 0))],
        out_specs=pl.BlockSpec((8, N), lambda i: (i, 0)),
        out_shape=jax.ShapeDtypeStruct((B, N), jnp.float32),
    )(x_t)

    bm3 = boxmax.reshape(B, NG, G)

    selbox = pl.pallas_call(
        _select_boxes_body,
        grid=(1,),
        in_specs=[pl.BlockSpec((B, NG, G), lambda i: (0, 0, 0))],
        out_specs=pl.BlockSpec((B, K), lambda i: (0, 0)),
        out_shape=jax.ShapeDtypeStruct((B, K), jnp.int32),
    )(bm3)

    bcol = jnp.arange(B, dtype=jnp.int32)[:, None] * jnp.ones(
        (1, K), jnp.int32)
    sel2 = jnp.stack([(bcol // 8).reshape(B * K),
                      (selbox // 128).reshape(B * K),
                      (bcol % 8).reshape(B * K)], axis=1)

    eslab, bslab = pl.pallas_call(
        _gather_body,
        grid_spec=pltpu.PrefetchScalarGridSpec(
            num_scalar_prefetch=1,
            grid=(B * K,),
            in_specs=[
                pl.BlockSpec((C, 8, 128),
                             lambda i, s: (0, s[i, 0], s[i, 1])),
                pl.BlockSpec((4, 8, 128),
                             lambda i, s: (0, s[i, 0], s[i, 1])),
            ],
            out_specs=[
                pl.BlockSpec((1, C, 128), lambda i, s: (i, 0, 0)),
                pl.BlockSpec((1, 4, 128), lambda i, s: (i, 0, 0)),
            ],
        ),
        out_shape=[jax.ShapeDtypeStruct((B * K, C, 128), jnp.float32),
                   jax.ShapeDtypeStruct((B * K, 4, 128), jnp.float32)],
    )(sel2, x_t, bx_t)

    out = pl.pallas_call(
        _finalize_body,
        grid=(1,),
        in_specs=[
            pl.BlockSpec((B, K), lambda i: (0, 0)),
            pl.BlockSpec((B * K, C, 128), lambda i: (0, 0, 0)),
            pl.BlockSpec((B * K, 4, 128), lambda i: (0, 0, 0)),
        ],
        out_specs=pl.BlockSpec((B, K, 14), lambda i: (0, 0, 0)),
        out_shape=jax.ShapeDtypeStruct((B, K, 14), jnp.float32),
    )(selbox, eslab, bslab)
    return out


# speedup vs baseline: 12.7427x; 3.2236x over previous
import jax
import jax.numpy as jnp
from jax import lax
from jax.experimental import pallas as pl
from jax.experimental.pallas import tpu as pltpu

K = 15
C = 10
N = 20000
B = 64
G = 160
NG = N // G
THRESHOLD = 0.7
NEG = -3.0e38


def _boxmax_body(x_ref, out_ref):
    acc = x_ref[0]
    for c in range(1, C):
        acc = jnp.maximum(acc, x_ref[c])
    out_ref[...] = acc


def _select_boxes_body(bm_ref, selbox_ref):
    bm = bm_ref[...]
    gmax = jnp.max(bm, axis=2)
    gid_iota = lax.broadcasted_iota(jnp.int32, (B, NG), 1)

    gids = []
    gm = gmax
    m = None
    for _ in range(K):
        m = jnp.max(gm, axis=1, keepdims=True)
        gi = jnp.min(jnp.where(gm == m, gid_iota, NG), axis=1,
                     keepdims=True)
        gids.append(gi)
        gm = jnp.where(gid_iota == gi, NEG, gm)
    t15 = m

    gsel = jnp.concatenate(gids, axis=1)
    g_iota = lax.broadcasted_iota(jnp.int32, (B, K, NG), 2)
    onehot = (g_iota == gsel[:, :, None]).astype(jnp.float32)
    cand = lax.dot_general(onehot, bm,
                           (((2,), (1,)), ((0,), (0,))),
                           precision=lax.Precision.HIGHEST,
                           preferred_element_type=jnp.float32)
    boxid = gsel[:, :, None] * G + lax.broadcasted_iota(
        jnp.int32, (B, K, G), 2)

    cand = jnp.where(cand >= t15[:, :, None], cand, NEG)

    big = jnp.int32(N)
    for k in range(K):
        m1 = jnp.max(cand, axis=1)
        m = jnp.max(m1, axis=1, keepdims=True)
        eq = cand == m[:, :, None]
        bi1 = jnp.min(jnp.where(eq, boxid, big), axis=1)
        bi = jnp.min(bi1, axis=1, keepdims=True)
        selbox_ref[:, pl.ds(k, 1)] = bi
        cand = jnp.where(boxid == bi[:, :, None], NEG, cand)


S = 16


def _gather_body(sel_s, *refs):
    i = pl.program_id(0)
    eo_ref = refs[2 * S]
    bo_ref = refs[2 * S + 1]
    sub_c = lax.broadcasted_iota(jnp.int32, (C, 8, 128), 1)
    sub_b = lax.broadcasted_iota(jnp.int32, (4, 8, 128), 1)
    for j in range(S):
        u = sel_s[i * S + j, 2]
        eo_ref[j] = jnp.sum(jnp.where(sub_c == u, refs[j][...], 0.0),
                            axis=1)
        bo_ref[j] = jnp.sum(jnp.where(sub_b == u, refs[S + j][...], 0.0),
                            axis=1)


def _finalize_body(selbox_ref, eslab_ref, bslab_ref, out_ref):
    selbox = selbox_ref[...]
    bcol = lax.broadcasted_iota(jnp.int32, (B, K), 0)
    l4 = (selbox % 128)[:, :, None, None]

    ew = eslab_ref[...].reshape(B, K, C, 128)
    lane4 = lax.broadcasted_iota(jnp.int32, (B, K, C, 128), 3)
    el3 = jnp.sum(jnp.where(lane4 == l4, ew, 0.0), axis=3)

    bw = bslab_ref[...].reshape(B, K, 4, 128)
    lb4 = lax.broadcasted_iota(jnp.int32, (B, K, 4, 128), 3)
    b43 = jnp.sum(jnp.where(lb4 == l4, bw, 0.0), axis=3)

    fi = selbox[:, :, None] * C + lax.broadcasted_iota(
        jnp.int32, (B, K, C), 2)

    vals = el3
    big = jnp.int32(N * C)
    out_rows = []
    keys = []
    for r in range(K):
        m = jnp.max(jnp.max(vals, axis=1), axis=1, keepdims=True)
        eq = vals == m[:, :, None]
        fr = jnp.min(jnp.min(jnp.where(eq, fi, big), axis=1),
                     axis=1, keepdims=True)
        vals = jnp.where(fi == fr[:, :, None], NEG, vals)
        score = 1.0 / (1.0 + jnp.exp(-m))
        use = jnp.where(score >= THRESHOLD, 1.0, 0.0)
        bi = fr // C
        pmask = (selbox == bi).astype(jnp.float32)[:, :, None]
        lrow = jnp.sum(pmask * el3, axis=1)
        brow = jnp.sum(pmask * b43, axis=1)
        prow = 1.0 / (1.0 + jnp.exp(-lrow))
        row = jnp.concatenate([brow, prow], axis=1) * use
        out_rows.append(row[:, None, :])
        keys.append(row[:, 0:1])
    player = jnp.concatenate(out_rows, axis=1)
    kmat = jnp.concatenate(keys, axis=1)

    slot_iota = lax.broadcasted_iota(jnp.int32, (B, K), 1)
    for i in range(K):
        m = jnp.max(kmat, axis=1, keepdims=True)
        p = jnp.min(jnp.where(kmat == m, slot_iota, K), axis=1,
                    keepdims=True)
        sel = (slot_iota == p).astype(jnp.float32)[:, :, None]
        out_ref[:, pl.ds(i, 1), :] = jnp.sum(sel * player, axis=1,
                                             keepdims=True)
        kmat = jnp.where(slot_iota == p, NEG, kmat)


@jax.jit
def kernel(logits, boxes):
    x_t = logits.transpose(2, 0, 1)
    bx_t = boxes.transpose(2, 0, 1)

    boxmax = pl.pallas_call(
        _boxmax_body,
        grid=(B // 8,),
        in_specs=[pl.BlockSpec((C, 8, N), lambda i: (0, i, 0))],
        out_specs=pl.BlockSpec((8, N), lambda i: (i, 0)),
        out_shape=jax.ShapeDtypeStruct((B, N), jnp.float32),
    )(x_t)

    bm3 = boxmax.reshape(B, NG, G)

    selbox = pl.pallas_call(
        _select_boxes_body,
        grid=(1,),
        in_specs=[pl.BlockSpec((B, NG, G), lambda i: (0, 0, 0))],
        out_specs=pl.BlockSpec((B, K), lambda i: (0, 0)),
        out_shape=jax.ShapeDtypeStruct((B, K), jnp.int32),
    )(bm3)

    bcol = jnp.arange(B, dtype=jnp.int32)[:, None] * jnp.ones(
        (1, K), jnp.int32)
    sel2 = jnp.stack([(bcol // 8).reshape(B * K),
                      (selbox // 128).reshape(B * K),
                      (bcol % 8).reshape(B * K)], axis=1)

    x_specs = [
        pl.BlockSpec((C, 8, 128),
                     lambda i, s, j=j: (0, s[i * S + j, 0], s[i * S + j, 1]))
        for j in range(S)]
    b_specs = [
        pl.BlockSpec((4, 8, 128),
                     lambda i, s, j=j: (0, s[i * S + j, 0], s[i * S + j, 1]))
        for j in range(S)]
    eslab, bslab = pl.pallas_call(
        _gather_body,
        grid_spec=pltpu.PrefetchScalarGridSpec(
            num_scalar_prefetch=1,
            grid=(B * K // S,),
            in_specs=x_specs + b_specs,
            out_specs=[
                pl.BlockSpec((S, C, 128), lambda i, s: (i, 0, 0)),
                pl.BlockSpec((S, 4, 128), lambda i, s: (i, 0, 0)),
            ],
        ),
        out_shape=[jax.ShapeDtypeStruct((B * K, C, 128), jnp.float32),
                   jax.ShapeDtypeStruct((B * K, 4, 128), jnp.float32)],
    )(sel2, *([x_t] * S), *([bx_t] * S))

    out = pl.pallas_call(
        _finalize_body,
        grid=(1,),
        in_specs=[
            pl.BlockSpec((B, K), lambda i: (0, 0)),
            pl.BlockSpec((B * K, C, 128), lambda i: (0, 0, 0)),
            pl.BlockSpec((B * K, 4, 128), lambda i: (0, 0, 0)),
        ],
        out_specs=pl.BlockSpec((B, K, 14), lambda i: (0, 0, 0)),
        out_shape=jax.ShapeDtypeStruct((B, K, 14), jnp.float32),
    )(selbox, eslab, bslab)
    return out


# speedup vs baseline: 12.7456x; 1.0002x over previous
import jax
import jax.numpy as jnp
from jax import lax
from jax.experimental import pallas as pl
from jax.experimental.pallas import tpu as pltpu

K = 15
C = 10
N = 20000
B = 64
G = 160
NG = N // G
THRESHOLD = 0.7
NEG = -3.0e38


def _boxmax_body(x_ref, out_ref):
    acc = x_ref[0]
    for c in range(1, C):
        acc = jnp.maximum(acc, x_ref[c])
    out_ref[...] = acc


def _select_boxes_body(bm_ref, selbox_ref):
    bm = bm_ref[...]
    gmax = jnp.max(bm, axis=2)
    gid_iota = lax.broadcasted_iota(jnp.int32, (B, NG), 1)

    gids = []
    gm = gmax
    m = None
    for _ in range(K):
        m = jnp.max(gm, axis=1, keepdims=True)
        gi = jnp.min(jnp.where(gm == m, gid_iota, NG), axis=1,
                     keepdims=True)
        gids.append(gi)
        gm = jnp.where(gid_iota == gi, NEG, gm)
    t15 = m

    gsel = jnp.concatenate(gids, axis=1)
    g_iota = lax.broadcasted_iota(jnp.int32, (B, K, NG), 2)
    onehot = (g_iota == gsel[:, :, None]).astype(jnp.float32)
    cand = lax.dot_general(onehot, bm,
                           (((2,), (1,)), ((0,), (0,))),
                           precision=lax.Precision.HIGHEST,
                           preferred_element_type=jnp.float32)
    boxid = gsel[:, :, None] * G + lax.broadcasted_iota(
        jnp.int32, (B, K, G), 2)

    cand = jnp.where(cand >= t15[:, :, None], cand, NEG)

    big = jnp.int32(N)
    for k in range(K):
        m1 = jnp.max(cand, axis=1)
        m = jnp.max(m1, axis=1, keepdims=True)
        eq = cand == m[:, :, None]
        bi1 = jnp.min(jnp.where(eq, boxid, big), axis=1)
        bi = jnp.min(bi1, axis=1, keepdims=True)
        selbox_ref[:, pl.ds(k, 1)] = bi
        cand = jnp.where(boxid == bi[:, :, None], NEG, cand)


S = 32


def _gather_body(sel_s, *refs):
    i = pl.program_id(0)
    eo_ref = refs[2 * S]
    bo_ref = refs[2 * S + 1]
    sub_c = lax.broadcasted_iota(jnp.int32, (C, 8, 128), 1)
    sub_b = lax.broadcasted_iota(jnp.int32, (4, 8, 128), 1)
    for j in range(S):
        u = sel_s[i * S + j, 2]
        eo_ref[j] = jnp.sum(jnp.where(sub_c == u, refs[j][...], 0.0),
                            axis=1)
        bo_ref[j] = jnp.sum(jnp.where(sub_b == u, refs[S + j][...], 0.0),
                            axis=1)


def _finalize_body(selbox_ref, eslab_ref, bslab_ref, out_ref):
    selbox = selbox_ref[...]
    bcol = lax.broadcasted_iota(jnp.int32, (B, K), 0)
    l4 = (selbox % 128)[:, :, None, None]

    ew = eslab_ref[...].reshape(B, K, C, 128)
    lane4 = lax.broadcasted_iota(jnp.int32, (B, K, C, 128), 3)
    el3 = jnp.sum(jnp.where(lane4 == l4, ew, 0.0), axis=3)

    bw = bslab_ref[...].reshape(B, K, 4, 128)
    lb4 = lax.broadcasted_iota(jnp.int32, (B, K, 4, 128), 3)
    b43 = jnp.sum(jnp.where(lb4 == l4, bw, 0.0), axis=3)

    fi = selbox[:, :, None] * C + lax.broadcasted_iota(
        jnp.int32, (B, K, C), 2)

    vals = el3
    big = jnp.int32(N * C)
    out_rows = []
    keys = []
    for r in range(K):
        m = jnp.max(jnp.max(vals, axis=1), axis=1, keepdims=True)
        eq = vals == m[:, :, None]
        fr = jnp.min(jnp.min(jnp.where(eq, fi, big), axis=1),
                     axis=1, keepdims=True)
        vals = jnp.where(fi == fr[:, :, None], NEG, vals)
        score = 1.0 / (1.0 + jnp.exp(-m))
        use = jnp.where(score >= THRESHOLD, 1.0, 0.0)
        bi = fr // C
        pmask = (selbox == bi).astype(jnp.float32)[:, :, None]
        lrow = jnp.sum(pmask * el3, axis=1)
        brow = jnp.sum(pmask * b43, axis=1)
        prow = 1.0 / (1.0 + jnp.exp(-lrow))
        row = jnp.concatenate([brow, prow], axis=1) * use
        out_rows.append(row[:, None, :])
        keys.append(row[:, 0:1])
    player = jnp.concatenate(out_rows, axis=1)
    kmat = jnp.concatenate(keys, axis=1)

    slot_iota = lax.broadcasted_iota(jnp.int32, (B, K), 1)
    for i in range(K):
        m = jnp.max(kmat, axis=1, keepdims=True)
        p = jnp.min(jnp.where(kmat == m, slot_iota, K), axis=1,
                    keepdims=True)
        sel = (slot_iota == p).astype(jnp.float32)[:, :, None]
        out_ref[:, pl.ds(i, 1), :] = jnp.sum(sel * player, axis=1,
                                             keepdims=True)
        kmat = jnp.where(slot_iota == p, NEG, kmat)


@jax.jit
def kernel(logits, boxes):
    x_t = logits.transpose(2, 0, 1)
    bx_t = boxes.transpose(2, 0, 1)

    boxmax = pl.pallas_call(
        _boxmax_body,
        grid=(B // 8,),
        in_specs=[pl.BlockSpec((C, 8, N), lambda i: (0, i, 0))],
        out_specs=pl.BlockSpec((8, N), lambda i: (i, 0)),
        out_shape=jax.ShapeDtypeStruct((B, N), jnp.float32),
    )(x_t)

    bm3 = boxmax.reshape(B, NG, G)

    selbox = pl.pallas_call(
        _select_boxes_body,
        grid=(1,),
        in_specs=[pl.BlockSpec((B, NG, G), lambda i: (0, 0, 0))],
        out_specs=pl.BlockSpec((B, K), lambda i: (0, 0)),
        out_shape=jax.ShapeDtypeStruct((B, K), jnp.int32),
    )(bm3)

    bcol = jnp.arange(B, dtype=jnp.int32)[:, None] * jnp.ones(
        (1, K), jnp.int32)
    sel2 = jnp.stack([(bcol // 8).reshape(B * K),
                      (selbox // 128).reshape(B * K),
                      (bcol % 8).reshape(B * K)], axis=1)

    x_specs = [
        pl.BlockSpec((C, 8, 128),
                     lambda i, s, j=j: (0, s[i * S + j, 0], s[i * S + j, 1]))
        for j in range(S)]
    b_specs = [
        pl.BlockSpec((4, 8, 128),
                     lambda i, s, j=j: (0, s[i * S + j, 0], s[i * S + j, 1]))
        for j in range(S)]
    eslab, bslab = pl.pallas_call(
        _gather_body,
        grid_spec=pltpu.PrefetchScalarGridSpec(
            num_scalar_prefetch=1,
            grid=(B * K // S,),
            in_specs=x_specs + b_specs,
            out_specs=[
                pl.BlockSpec((S, C, 128), lambda i, s: (i, 0, 0)),
                pl.BlockSpec((S, 4, 128), lambda i, s: (i, 0, 0)),
            ],
        ),
        out_shape=[jax.ShapeDtypeStruct((B * K, C, 128), jnp.float32),
                   jax.ShapeDtypeStruct((B * K, 4, 128), jnp.float32)],
    )(sel2, *([x_t] * S), *([bx_t] * S))

    out = pl.pallas_call(
        _finalize_body,
        grid=(1,),
        in_specs=[
            pl.BlockSpec((B, K), lambda i: (0, 0)),
            pl.BlockSpec((B * K, C, 128), lambda i: (0, 0, 0)),
            pl.BlockSpec((B * K, 4, 128), lambda i: (0, 0, 0)),
        ],
        out_specs=pl.BlockSpec((B, K, 14), lambda i: (0, 0, 0)),
        out_shape=jax.ShapeDtypeStruct((B, K, 14), jnp.float32),
    )(selbox, eslab, bslab)
    return out
